# baseline (speedup 1.0000x reference)
import functools
import math

import jax
import jax.numpy as jnp
from jax import lax
from jax.experimental import pallas as pl
from jax.experimental.pallas import tpu as pltpu
from jax.experimental.pallas import tpu_sc as plsc

VOCAB = 100000
D_EMB = 300
D_PAD = 304
SCALE = math.sqrt(float(D_EMB))

NC, NS = 2, 16
NW = NC * NS
CH = 64
NRB = 4
NPB = 2
LANES = 16
FULL_VREGS = D_EMB // LANES
TAIL = D_EMB - FULL_VREGS * LANES


def _emb_body(x_hbm, table_hbm, out_hbm, idx_v, rows0, rows1, rows2, rows3,
              pack0, pack1, g0, g1, g2, g3, w0, w1, nch):
  wid = lax.axis_index("s") * NC + lax.axis_index("c")
  pltpu.sync_copy(x_hbm.at[wid], idx_v)

  rows_bufs = (rows0, rows1, rows2, rows3)
  g_sems = (g0, g1, g2, g3)
  pack_bufs = (pack0, pack1)
  w_sems = (w0, w1)

  iota = lax.broadcasted_iota(jnp.int32, (LANES,), 0)
  offs = [iota + k * LANES for k in range(FULL_VREGS)]
  offs.append(jnp.where(iota < TAIL, iota, TAIL - 1) + FULL_VREGS * LANES)
  tail_mask = iota < TAIL

  def out_slice(i):
    return out_hbm.at[pl.ds((wid * nch + i) * CH * D_EMB, CH * D_EMB)]

  def scale_chunk(rows_v, pack_v):
    def row(r, _):
      base = r * D_EMB
      for k in range(FULL_VREGS):
        v = rows_v[r, pl.ds(k * LANES, LANES)] * SCALE
        plsc.store_scatter(pack_v, [base + offs[k]], v)
      v = rows_v[r, pl.ds(FULL_VREGS * LANES, LANES)] * SCALE
      plsc.store_scatter(pack_v, [base + offs[FULL_VREGS]], v, mask=tail_mask)
      return 0

    lax.fori_loop(0, CH, row, 0)

  pltpu.async_copy(table_hbm.at[idx_v.at[0]], rows0, g0)

  def quad(j, _):
    for db in range(NRB):
      i = NRB * j + db
      rows_v, gs = rows_bufs[db], g_sems[db]
      pb = db % NPB
      pack_v, ws = pack_bufs[pb], w_sems[pb]
      nb = (db + 1) % NRB

      @pl.when(i + 1 < nch)
      def _():
        pltpu.async_copy(table_hbm.at[idx_v.at[i + 1]], rows_bufs[nb],
                         g_sems[nb])

      pltpu.make_async_copy(table_hbm.at[idx_v.at[i]], rows_v, gs).wait()

      @pl.when(i >= NPB)
      def _():
        pltpu.make_async_copy(pack_v, out_slice(i - NPB), ws).wait()

      scale_chunk(rows_v, pack_v)
      pltpu.async_copy(pack_v, out_slice(i), ws)
    return 0

  lax.fori_loop(0, nch // NRB, quad, 0)
  pltpu.make_async_copy(pack0, out_slice(nch - 2), w0).wait()
  pltpu.make_async_copy(pack1, out_slice(nch - 1), w1).wait()


def kernel(x, table):
  B, L = x.shape
  n_total = B * L
  assert n_total % (NW * CH) == 0
  nch = n_total // (NW * CH)
  assert nch % NRB == 0

  mesh = plsc.VectorSubcoreMesh(core_axis_name="c", subcore_axis_name="s")
  k = pl.kernel(
      functools.partial(_emb_body, nch=nch),
      out_type=jax.ShapeDtypeStruct((n_total * D_EMB,), jnp.float32),
      mesh=mesh,
      compiler_params=pltpu.CompilerParams(
          use_tc_tiling_on_sc=False, needs_layout_passes=False),
      scratch_types=[
          pltpu.VMEM((nch, CH), jnp.int32),
          pltpu.VMEM((CH, D_PAD), jnp.float32),
          pltpu.VMEM((CH, D_PAD), jnp.float32),
          pltpu.VMEM((CH, D_PAD), jnp.float32),
          pltpu.VMEM((CH, D_PAD), jnp.float32),
          pltpu.VMEM((CH * D_EMB,), jnp.float32),
          pltpu.VMEM((CH * D_EMB,), jnp.float32),
          pltpu.SemaphoreType.DMA,
          pltpu.SemaphoreType.DMA,
          pltpu.SemaphoreType.DMA,
          pltpu.SemaphoreType.DMA,
          pltpu.SemaphoreType.DMA,
          pltpu.SemaphoreType.DMA,
      ],
  )
  x2 = x.reshape(NW, nch, CH)
  table_pad = jnp.pad(table, ((0, 0), (0, D_PAD - D_EMB)))
  out = k(x2, table_pad)
  return out.reshape(B, L, D_EMB)


# speedup vs baseline: 1.3686x; 1.3686x over previous
import functools
import math

import jax
import jax.numpy as jnp
from jax import lax
from jax.experimental import pallas as pl
from jax.experimental.pallas import tpu as pltpu
from jax.experimental.pallas import tpu_sc as plsc

VOCAB = 100000
D_EMB = 300
D_PAD = 304
SCALE = math.sqrt(float(D_EMB))

NC, NS = 2, 16
NW = NC * NS
CH = 64
NRB = 4
NPB = 2
LANES = 16
FULL_VREGS = D_EMB // LANES
TAIL = D_EMB - FULL_VREGS * LANES


def _emb_body(x_hbm, table_hbm, out_hbm, idx_v, rows0, rows1, rows2, rows3,
              pack0, pack1, g0, g1, g2, g3, w0, w1, nch):
  wid = lax.axis_index("s") * NC + lax.axis_index("c")
  pltpu.sync_copy(x_hbm.at[wid], idx_v)

  rows_bufs = (rows0, rows1, rows2, rows3)
  g_sems = (g0, g1, g2, g3)
  pack_bufs = (pack0, pack1)
  w_sems = (w0, w1)

  iota = lax.broadcasted_iota(jnp.int32, (LANES,), 0)
  offs = [iota + k * LANES for k in range(FULL_VREGS)]
  offs.append(jnp.where(iota < TAIL, iota, TAIL - 1) + FULL_VREGS * LANES)
  tail_mask = iota < TAIL

  def out_slice(i):
    return out_hbm.at[pl.ds((wid * nch + i) * CH * D_EMB, CH * D_EMB)]

  def scale_chunk(rows_v, pack_v):
    def row(r, _):
      base = r * D_EMB
      for k in range(FULL_VREGS):
        plsc.store_scatter(pack_v, [base + offs[k]],
                           rows_v[r, pl.ds(k * LANES, LANES)])
      plsc.store_scatter(pack_v, [base + offs[FULL_VREGS]],
                         rows_v[r, pl.ds(FULL_VREGS * LANES, LANES)],
                         mask=tail_mask)
      return 0

    lax.fori_loop(0, CH, row, 0)

  pltpu.async_copy(table_hbm.at[idx_v.at[0]], rows0, g0)

  def quad(j, _):
    for db in range(NRB):
      i = NRB * j + db
      rows_v, gs = rows_bufs[db], g_sems[db]
      pb = db % NPB
      pack_v, ws = pack_bufs[pb], w_sems[pb]
      nb = (db + 1) % NRB

      @pl.when(i + 1 < nch)
      def _():
        pltpu.async_copy(table_hbm.at[idx_v.at[i + 1]], rows_bufs[nb],
                         g_sems[nb])

      pltpu.make_async_copy(table_hbm.at[idx_v.at[i]], rows_v, gs).wait()

      @pl.when(i >= NPB)
      def _():
        pltpu.make_async_copy(pack_v, out_slice(i - NPB), ws).wait()

      scale_chunk(rows_v, pack_v)
      pltpu.async_copy(pack_v, out_slice(i), ws)
    return 0

  lax.fori_loop(0, nch // NRB, quad, 0)
  pltpu.make_async_copy(pack0, out_slice(nch - 2), w0).wait()
  pltpu.make_async_copy(pack1, out_slice(nch - 1), w1).wait()


V_BLK = 512


def _prep_body(tT_ref, out_ref):
  blk = tT_ref[...]
  xp = jnp.pad(blk, ((0, D_PAD - D_EMB), (0, 0)))
  out_ref[...] = jnp.transpose(xp, (1, 0)) * SCALE


def _prep_table(table):
  tT = table.T
  grid = (VOCAB + V_BLK - 1) // V_BLK
  return pl.pallas_call(
      _prep_body,
      grid=(grid,),
      in_specs=[pl.BlockSpec((D_EMB, V_BLK), lambda i: (0, i))],
      out_specs=pl.BlockSpec((V_BLK, D_PAD), lambda i: (i, 0)),
      out_shape=jax.ShapeDtypeStruct((VOCAB, D_PAD), jnp.float32),
  )(tT)


def kernel(x, table):
  B, L = x.shape
  n_total = B * L
  assert n_total % (NW * CH) == 0
  nch = n_total // (NW * CH)
  assert nch % NRB == 0

  mesh = plsc.VectorSubcoreMesh(core_axis_name="c", subcore_axis_name="s")
  k = pl.kernel(
      functools.partial(_emb_body, nch=nch),
      out_type=jax.ShapeDtypeStruct((n_total * D_EMB,), jnp.float32),
      mesh=mesh,
      compiler_params=pltpu.CompilerParams(
          use_tc_tiling_on_sc=False, needs_layout_passes=False),
      scratch_types=[
          pltpu.VMEM((nch, CH), jnp.int32),
          pltpu.VMEM((CH, D_PAD), jnp.float32),
          pltpu.VMEM((CH, D_PAD), jnp.float32),
          pltpu.VMEM((CH, D_PAD), jnp.float32),
          pltpu.VMEM((CH, D_PAD), jnp.float32),
          pltpu.VMEM((CH * D_EMB,), jnp.float32),
          pltpu.VMEM((CH * D_EMB,), jnp.float32),
          pltpu.SemaphoreType.DMA,
          pltpu.SemaphoreType.DMA,
          pltpu.SemaphoreType.DMA,
          pltpu.SemaphoreType.DMA,
          pltpu.SemaphoreType.DMA,
          pltpu.SemaphoreType.DMA,
      ],
  )
  x2 = x.reshape(NW, nch, CH)
  table_pad = _prep_table(table)
  out = k(x2, table_pad)
  return out.reshape(B, L, D_EMB)


# speedup vs baseline: 1.9759x; 1.4438x over previous
import functools
import math

import jax
import jax.numpy as jnp
from jax import lax
from jax.experimental import pallas as pl
from jax.experimental.pallas import tpu as pltpu
from jax.experimental.pallas import tpu_sc as plsc

VOCAB = 100000
D_EMB = 300
D_PAD = 304
SCALE = math.sqrt(float(D_EMB))

NC, NS = 2, 16
NW = NC * NS
NB = 2
NRB = 4
V_BLK = 512


def _emb_body(x_hbm, table_hbm, out_hbm, idx_v, rows0, rows1, rows2, rows3,
              g0, g1, g2, g3, w0, w1, w2, w3, nch, L):
  wid = lax.axis_index("s") * NC + lax.axis_index("c")
  pltpu.sync_copy(x_hbm.at[wid], idx_v)

  rows_bufs = (rows0, rows1, rows2, rows3)
  g_sems = (g0, g1, g2, g3)
  w_sems = (w0, w1, w2, w3)

  def out_box(i):
    return out_hbm.at[pl.ds((wid * nch + i) * NB, NB)]

  def start_gather(i, db):
    for m in range(NB):
      pltpu.async_copy(table_hbm.at[idx_v.at[i, m]], rows_bufs[db].at[m],
                       g_sems[db])

  def wait_gather(i, db):
    for m in range(NB):
      pltpu.make_async_copy(table_hbm.at[idx_v.at[i, m]],
                            rows_bufs[db].at[m], g_sems[db]).wait()

  start_gather(0, 0)

  def quad(j, _):
    for db in range(NRB):
      i = NRB * j + db
      rows_v = rows_bufs[db]
      nb = (db + 1) % NRB

      @pl.when(i + 1 < nch)
      def _():
        @pl.when(i + 1 >= NRB)
        def _():
          pltpu.make_async_copy(rows_bufs[nb], out_box(i + 1 - NRB),
                                w_sems[nb]).wait()
        start_gather(i + 1, nb)

      wait_gather(i, db)
      pltpu.async_copy(rows_v, out_box(i), w_sems[db])
    return 0

  lax.fori_loop(0, nch // NRB, quad, 0)
  for db in range(NRB):
    pltpu.make_async_copy(rows_bufs[db], out_box(nch - NRB + db),
                          w_sems[db]).wait()


def _prep_body(tT_ref, out_ref):
  blk = tT_ref[...]
  xp = jnp.pad(blk, ((0, D_PAD - D_EMB), (0, 0)))
  out_ref[...] = jnp.transpose(xp, (1, 0)) * SCALE


def _prep_table(table):
  tT = table.T
  grid = (VOCAB + V_BLK - 1) // V_BLK
  return pl.pallas_call(
      _prep_body,
      grid=(grid,),
      in_specs=[pl.BlockSpec((D_EMB, V_BLK), lambda i: (0, i))],
      out_specs=pl.BlockSpec((V_BLK, D_PAD), lambda i: (i, 0)),
      out_shape=jax.ShapeDtypeStruct((VOCAB, D_PAD), jnp.float32),
  )(tT)


def kernel(x, table):
  B, L = x.shape
  assert B % (NW * NB) == 0
  nch = B // (NW * NB)
  assert nch % NRB == 0

  mesh = plsc.VectorSubcoreMesh(core_axis_name="c", subcore_axis_name="s")
  k = pl.kernel(
      functools.partial(_emb_body, nch=nch, L=L),
      out_type=jax.ShapeDtypeStruct((B, L, D_PAD), jnp.float32),
      mesh=mesh,
      compiler_params=pltpu.CompilerParams(
          use_tc_tiling_on_sc=False, needs_layout_passes=False),
      scratch_types=[
          pltpu.VMEM((nch, NB, L), jnp.int32),
          pltpu.VMEM((NB, L, D_PAD), jnp.float32),
          pltpu.VMEM((NB, L, D_PAD), jnp.float32),
          pltpu.VMEM((NB, L, D_PAD), jnp.float32),
          pltpu.VMEM((NB, L, D_PAD), jnp.float32),
          pltpu.SemaphoreType.DMA,
          pltpu.SemaphoreType.DMA,
          pltpu.SemaphoreType.DMA,
          pltpu.SemaphoreType.DMA,
          pltpu.SemaphoreType.DMA,
          pltpu.SemaphoreType.DMA,
          pltpu.SemaphoreType.DMA,
          pltpu.SemaphoreType.DMA,
      ],
  )
  x2 = x.reshape(NW, nch, NB, L)
  table_pad = _prep_table(table)
  out = k(x2, table_pad)
  return out[:, :, :D_EMB]
